# baseline (speedup 1.0000x reference)
import jax
import jax.numpy as jnp
from jax import lax
from jax.experimental import pallas as pl
from jax.experimental.pallas import tpu as pltpu

_EPS = 1e-5
_N, _B, _C, _H, _W = 8, 4, 256, 64, 64
_NTOT = _N + 1
_HW = _H * _W
_INV_CHW = 1.0 / (_C * _H * _W)
_LOG2E = 1.4426950408889634
_HCHUNK = 8


def _stats_and_pw(ref3, wmat_ref, pw_ref):
    acc_s = jnp.zeros((_HCHUNK * _W, 128), jnp.float32)
    acc_q = jnp.zeros((_HCHUNK * _W, _C), jnp.float32)
    for h in range(0, _H, _HCHUNK):
        vc = ref3[h:h + _HCHUNK].reshape(_HCHUNK * _W, _C)
        vb = vc.astype(jnp.bfloat16)
        r = jnp.dot(vb, wmat_ref[...], preferred_element_type=jnp.float32)
        pw_ref[h * _W:(h + _HCHUNK) * _W] = r[:, :128]
        acc_s = acc_s + r[:, 128:]
        acc_q = acc_q + vc * vc
    mean = jnp.sum(acc_s) * (_INV_CHW / 128.0)
    var = jnp.sum(acc_q) * _INV_CHW - mean * mean
    return mean, lax.rsqrt(var + _EPS)


def _body(blocks_ref, x_ref, wmat_ref, s_ref, out_ref, pw_ref, l_ref):
    n = pl.program_id(1)
    s_sum = s_ref[0, 0]

    @pl.when(n == 0)
    def _init():
        mean, rstd = _stats_and_pw(x_ref.at[0], wmat_ref, pw_ref)
        rstd = rstd * _LOG2E
        shift = mean * s_sum
        for h in range(0, _H, _HCHUNK):
            r0, r1 = h * _W, (h + _HCHUNK) * _W
            p = jnp.exp2((pw_ref[r0:r1] - shift) * rstd)
            l_ref[r0:r1] = p
            p3 = pltpu.repeat(p, 2, axis=1).reshape(_HCHUNK, _W, _C)
            out_ref[0, h:h + _HCHUNK] = x_ref[0, h:h + _HCHUNK] * p3

    @pl.when((n > 0) & (n < _NTOT - 1))
    def _update():
        mean, rstd = _stats_and_pw(blocks_ref.at[0, 0], wmat_ref, pw_ref)
        rstd = rstd * _LOG2E
        shift = mean * s_sum
        for h in range(0, _H, _HCHUNK):
            r0, r1 = h * _W, (h + _HCHUNK) * _W
            p = jnp.exp2((pw_ref[r0:r1] - shift) * rstd)
            l_ref[r0:r1] = l_ref[r0:r1] + p
            p3 = pltpu.repeat(p, 2, axis=1).reshape(_HCHUNK, _W, _C)
            out_ref[0, h:h + _HCHUNK] = (out_ref[0, h:h + _HCHUNK]
                                         + blocks_ref[0, 0, h:h + _HCHUNK] * p3)

    @pl.when(n == _NTOT - 1)
    def _last():
        mean, rstd = _stats_and_pw(blocks_ref.at[0, 0], wmat_ref, pw_ref)
        rstd = rstd * _LOG2E
        shift = mean * s_sum
        for h in range(0, _H, _HCHUNK):
            r0, r1 = h * _W, (h + _HCHUNK) * _W
            p = jnp.exp2((pw_ref[r0:r1] - shift) * rstd)
            inv_l = 1.0 / (l_ref[r0:r1] + p)
            p3 = pltpu.repeat(p * inv_l, 2, axis=1).reshape(_HCHUNK, _W, _C)
            i3 = pltpu.repeat(inv_l, 2, axis=1).reshape(_HCHUNK, _W, _C)
            out_ref[0, h:h + _HCHUNK] = (out_ref[0, h:h + _HCHUNK] * i3
                                         + blocks_ref[0, 0, h:h + _HCHUNK] * p3)


def kernel(blocks, x, w, gn_weight, gn_bias):
    del gn_bias
    weff = (w * gn_weight).astype(jnp.float32)
    bt = jnp.transpose(blocks, (0, 1, 3, 4, 2))
    xt = jnp.transpose(x, (0, 2, 3, 1))
    wmat = jnp.concatenate(
        [jnp.broadcast_to(weff[:, None], (_C, 128)),
         jnp.ones((_C, 128), jnp.float32)], axis=1).astype(jnp.bfloat16)
    s_sum = jnp.sum(weff).reshape(1, 1)

    out_t = pl.pallas_call(
        _body,
        grid=(_B, _NTOT),
        in_specs=[
            pl.BlockSpec((1, 1, _H, _W, _C),
                         lambda b, n: (jnp.maximum(n - 1, 0), b, 0, 0, 0)),
            pl.BlockSpec((1, _H, _W, _C), lambda b, n: (b, 0, 0, 0)),
            pl.BlockSpec((_C, 256), lambda b, n: (0, 0)),
            pl.BlockSpec(memory_space=pltpu.SMEM),
        ],
        out_specs=pl.BlockSpec((1, _H, _W, _C), lambda b, n: (b, 0, 0, 0)),
        out_shape=jax.ShapeDtypeStruct((_B, _H, _W, _C), jnp.float32),
        scratch_shapes=[
            pltpu.VMEM((_HW, 128), jnp.float32),
            pltpu.VMEM((_HW, 128), jnp.float32),
        ],
        compiler_params=pltpu.CompilerParams(
            dimension_semantics=("parallel", "arbitrary"),
            vmem_limit_bytes=100 * 1024 * 1024,
        ),
    )(bt, xt, wmat, s_sum)
    return jnp.transpose(out_t, (0, 3, 1, 2))


# speedup vs baseline: 1.0588x; 1.0588x over previous
import jax
import jax.numpy as jnp
from jax import lax
from jax.experimental import pallas as pl
from jax.experimental.pallas import tpu as pltpu

_EPS = 1e-5
_N, _B, _C, _H, _W = 8, 4, 256, 64, 64
_NTOT = _N + 1
_HW = _H * _W
_INV_CHW = 1.0 / (_C * _H * _W)
_LOG2E = 1.4426950408889634
_HCHUNK = 8


def _stats_and_pw(ref3, wmat_ref, pw_ref):
    acc_s = jnp.zeros((_HCHUNK * _W, _C), jnp.float32)
    acc_q = jnp.zeros((_HCHUNK * _W, _C), jnp.float32)
    for h in range(0, _H, _HCHUNK):
        vc = ref3[h:h + _HCHUNK].reshape(_HCHUNK * _W, _C)
        acc_s = acc_s + vc
        acc_q = acc_q + vc * vc
        pw_ref[h * _W:(h + _HCHUNK) * _W] = jnp.dot(
            vc.astype(jnp.bfloat16), wmat_ref[...],
            preferred_element_type=jnp.float32)
    mean = jnp.sum(acc_s) * _INV_CHW
    var = jnp.sum(acc_q) * _INV_CHW - mean * mean
    return mean, lax.rsqrt(var + _EPS) * _LOG2E


def _body(blocks_ref, x_ref, wmat_ref, s_ref, out_ref, pw_ref, l_ref):
    n = pl.program_id(1)
    s_sum = s_ref[0, 0]

    @pl.when(n == 0)
    def _init():
        mean, rstd = _stats_and_pw(x_ref.at[0], wmat_ref, pw_ref)
        shift = mean * s_sum
        for h in range(0, _H, _HCHUNK):
            r0, r1 = h * _W, (h + _HCHUNK) * _W
            p = jnp.exp2((pw_ref[r0:r1] - shift) * rstd)
            l_ref[r0:r1] = p
            p3 = pltpu.repeat(p, 2, axis=1).reshape(_HCHUNK, _W, _C)
            out_ref[0, h:h + _HCHUNK] = x_ref[0, h:h + _HCHUNK] * p3

    @pl.when((n > 0) & (n < _NTOT - 1))
    def _update():
        mean, rstd = _stats_and_pw(blocks_ref.at[0, 0], wmat_ref, pw_ref)
        shift = mean * s_sum
        for h in range(0, _H, _HCHUNK):
            r0, r1 = h * _W, (h + _HCHUNK) * _W
            p = jnp.exp2((pw_ref[r0:r1] - shift) * rstd)
            l_ref[r0:r1] = l_ref[r0:r1] + p
            p3 = pltpu.repeat(p, 2, axis=1).reshape(_HCHUNK, _W, _C)
            out_ref[0, h:h + _HCHUNK] = (out_ref[0, h:h + _HCHUNK]
                                         + blocks_ref[0, 0, h:h + _HCHUNK] * p3)

    @pl.when(n == _NTOT - 1)
    def _last():
        mean, rstd = _stats_and_pw(blocks_ref.at[0, 0], wmat_ref, pw_ref)
        shift = mean * s_sum
        for h in range(0, _H, _HCHUNK):
            r0, r1 = h * _W, (h + _HCHUNK) * _W
            p = jnp.exp2((pw_ref[r0:r1] - shift) * rstd)
            inv_l = 1.0 / (l_ref[r0:r1] + p)
            p3 = pltpu.repeat(p * inv_l, 2, axis=1).reshape(_HCHUNK, _W, _C)
            i3 = pltpu.repeat(inv_l, 2, axis=1).reshape(_HCHUNK, _W, _C)
            out_ref[0, h:h + _HCHUNK] = (out_ref[0, h:h + _HCHUNK] * i3
                                         + blocks_ref[0, 0, h:h + _HCHUNK] * p3)


def kernel(blocks, x, w, gn_weight, gn_bias):
    del gn_bias
    weff = (w * gn_weight).astype(jnp.float32)
    bt = jnp.transpose(blocks, (0, 1, 3, 4, 2))
    xt = jnp.transpose(x, (0, 2, 3, 1))
    wmat = jnp.broadcast_to(weff[:, None], (_C, 128)).astype(jnp.bfloat16)
    s_sum = jnp.sum(weff).reshape(1, 1)

    out_t = pl.pallas_call(
        _body,
        grid=(_B, _NTOT),
        in_specs=[
            pl.BlockSpec((1, 1, _H, _W, _C),
                         lambda b, n: (jnp.maximum(n - 1, 0), b, 0, 0, 0)),
            pl.BlockSpec((1, _H, _W, _C), lambda b, n: (b, 0, 0, 0)),
            pl.BlockSpec((_C, 128), lambda b, n: (0, 0)),
            pl.BlockSpec(memory_space=pltpu.SMEM),
        ],
        out_specs=pl.BlockSpec((1, _H, _W, _C), lambda b, n: (b, 0, 0, 0)),
        out_shape=jax.ShapeDtypeStruct((_B, _H, _W, _C), jnp.float32),
        scratch_shapes=[
            pltpu.VMEM((_HW, 128), jnp.float32),
            pltpu.VMEM((_HW, 128), jnp.float32),
        ],
        compiler_params=pltpu.CompilerParams(
            dimension_semantics=("parallel", "arbitrary"),
            vmem_limit_bytes=100 * 1024 * 1024,
        ),
    )(bt, xt, wmat, s_sum)
    return jnp.transpose(out_t, (0, 3, 1, 2))


# speedup vs baseline: 1.0639x; 1.0048x over previous
import jax
import jax.numpy as jnp
from jax import lax
from jax.experimental import pallas as pl
from jax.experimental.pallas import tpu as pltpu

_EPS = 1e-5
_N, _B, _C, _H, _W = 8, 4, 256, 64, 64
_NTOT = _N + 1
_HW = _H * _W
_INV_CHW = 1.0 / (_C * _H * _W)
_LOG2E = 1.4426950408889634
_HCHUNK = 8


def _stats_and_pw(ref3, wmat_ref, pw_ref):
    acc_s = jnp.zeros((_HCHUNK * _W, _C), jnp.float32)
    acc_q = jnp.zeros((_HCHUNK * _W, _C), jnp.float32)
    for h in range(0, _H, _HCHUNK):
        vc = ref3[h:h + _HCHUNK].reshape(_HCHUNK * _W, _C)
        acc_s = acc_s + vc
        acc_q = acc_q + vc * vc
        pw_ref[h * _W:(h + _HCHUNK) * _W] = jnp.dot(
            vc.astype(jnp.bfloat16), wmat_ref[...],
            preferred_element_type=jnp.float32)
    mean = jnp.sum(acc_s) * _INV_CHW
    var = jnp.sum(acc_q) * _INV_CHW - mean * mean
    return mean, lax.rsqrt(var + _EPS) * _LOG2E


def _body(blocks_ref, x_ref, wmat_ref, s_ref, out_ref, pw_ref, l_ref):
    n = pl.program_id(1)
    s_sum = s_ref[0, 0]

    @pl.when(n == 0)
    def _init():
        mean, rstd = _stats_and_pw(x_ref.at[0], wmat_ref, pw_ref)
        shift = mean * s_sum
        for h in range(0, _H, _HCHUNK):
            r0, r1 = h * _W, (h + _HCHUNK) * _W
            p = jnp.exp2((pw_ref[r0:r1] - shift) * rstd)
            l_ref[r0:r1] = p
            p3 = pltpu.repeat(p, 2, axis=1).reshape(_HCHUNK, _W, _C)
            out_ref[0, h:h + _HCHUNK] = x_ref[0, h:h + _HCHUNK] * p3

    @pl.when((n > 0) & (n < _NTOT - 1))
    def _update():
        mean, rstd = _stats_and_pw(blocks_ref.at[0, 0], wmat_ref, pw_ref)
        shift = mean * s_sum
        for h in range(0, _H, _HCHUNK):
            r0, r1 = h * _W, (h + _HCHUNK) * _W
            p = jnp.exp2((pw_ref[r0:r1] - shift) * rstd)
            l_ref[r0:r1] = l_ref[r0:r1] + p
            p3 = pltpu.repeat(p, 2, axis=1).reshape(_HCHUNK, _W, _C)
            out_ref[0, h:h + _HCHUNK] = (out_ref[0, h:h + _HCHUNK]
                                         + blocks_ref[0, 0, h:h + _HCHUNK] * p3)

    @pl.when(n == _NTOT - 1)
    def _last():
        mean, rstd = _stats_and_pw(blocks_ref.at[0, 0], wmat_ref, pw_ref)
        shift = mean * s_sum
        for h in range(0, _H, _HCHUNK):
            r0, r1 = h * _W, (h + _HCHUNK) * _W
            p = jnp.exp2((pw_ref[r0:r1] - shift) * rstd)
            inv_l = 1.0 / (l_ref[r0:r1] + p)
            p3 = pltpu.repeat(p * inv_l, 2, axis=1).reshape(_HCHUNK, _W, _C)
            i3 = pltpu.repeat(inv_l, 2, axis=1).reshape(_HCHUNK, _W, _C)
            out_ref[0, h:h + _HCHUNK] = (out_ref[0, h:h + _HCHUNK] * i3
                                         + blocks_ref[0, 0, h:h + _HCHUNK] * p3)


def kernel(blocks, x, w, gn_weight, gn_bias):
    del gn_bias
    weff = (w * gn_weight).astype(jnp.float32)
    bt = jnp.transpose(blocks, (0, 1, 3, 4, 2))
    xt = jnp.transpose(x, (0, 2, 3, 1))
    wmat = jnp.broadcast_to(weff[:, None], (_C, 128)).astype(jnp.bfloat16)
    s_sum = jnp.sum(weff).reshape(1, 1)

    out_t = pl.pallas_call(
        _body,
        grid=(_B, _NTOT),
        in_specs=[
            pl.BlockSpec((1, 1, _H, _W, _C),
                         lambda b, n: (jnp.maximum(n - 1, 0), b, 0, 0, 0)),
            pl.BlockSpec((1, _H, _W, _C),
                         lambda b, n: (jnp.minimum(b + (n >= 5), _B - 1),
                                       0, 0, 0)),
            pl.BlockSpec((_C, 128), lambda b, n: (0, 0)),
            pl.BlockSpec(memory_space=pltpu.SMEM),
        ],
        out_specs=pl.BlockSpec((1, _H, _W, _C), lambda b, n: (b, 0, 0, 0)),
        out_shape=jax.ShapeDtypeStruct((_B, _H, _W, _C), jnp.float32),
        scratch_shapes=[
            pltpu.VMEM((_HW, 128), jnp.float32),
            pltpu.VMEM((_HW, 128), jnp.float32),
        ],
        compiler_params=pltpu.CompilerParams(
            dimension_semantics=("parallel", "arbitrary"),
            vmem_limit_bytes=100 * 1024 * 1024,
        ),
    )(bt, xt, wmat, s_sum)
    return jnp.transpose(out_t, (0, 3, 1, 2))


# speedup vs baseline: 1.1503x; 1.0812x over previous
import jax
import jax.numpy as jnp
from jax import lax
from jax.experimental import pallas as pl
from jax.experimental.pallas import tpu as pltpu

_EPS = 1e-5
_N, _B, _C, _H, _W = 8, 4, 256, 64, 64
_NSTEP = _N // 2 + 1
_HW = _H * _W
_INV_CHW = 1.0 / (_C * _H * _W)
_LOG2E = 1.4426950408889634
_HCHUNK = 8


def _stats_and_pw(ref3, wmat_ref, pw_ref):
    acc_s = jnp.zeros((_HCHUNK * _W, _C), jnp.float32)
    acc_q = jnp.zeros((_HCHUNK * _W, _C), jnp.float32)
    for h in range(0, _H, _HCHUNK):
        vc = ref3[h:h + _HCHUNK].reshape(_HCHUNK * _W, _C)
        acc_s = acc_s + vc
        acc_q = acc_q + vc * vc
        pw_ref[h * _W:(h + _HCHUNK) * _W] = jnp.dot(
            vc.astype(jnp.bfloat16), wmat_ref[...],
            preferred_element_type=jnp.float32)
    mean = jnp.sum(acc_s) * _INV_CHW
    var = jnp.sum(acc_q) * _INV_CHW - mean * mean
    return mean, lax.rsqrt(var + _EPS) * _LOG2E


def _apply(src3, wmat_ref, s_sum, out_ref, pw_ref, l_ref, mode):
    mean, rstd = _stats_and_pw(src3, wmat_ref, pw_ref)
    shift = mean * s_sum
    for h in range(0, _H, _HCHUNK):
        r0, r1 = h * _W, (h + _HCHUNK) * _W
        p = jnp.exp2((pw_ref[r0:r1] - shift) * rstd)
        if mode == "init":
            l_ref[r0:r1] = p
            p3 = pltpu.repeat(p, 2, axis=1).reshape(_HCHUNK, _W, _C)
            out_ref[0, h:h + _HCHUNK] = src3[h:h + _HCHUNK] * p3
        elif mode == "mid":
            l_ref[r0:r1] = l_ref[r0:r1] + p
            p3 = pltpu.repeat(p, 2, axis=1).reshape(_HCHUNK, _W, _C)
            out_ref[0, h:h + _HCHUNK] = (out_ref[0, h:h + _HCHUNK]
                                         + src3[h:h + _HCHUNK] * p3)
        else:
            inv_l = 1.0 / (l_ref[r0:r1] + p)
            p3 = pltpu.repeat(p * inv_l, 2, axis=1).reshape(_HCHUNK, _W, _C)
            i3 = pltpu.repeat(inv_l, 2, axis=1).reshape(_HCHUNK, _W, _C)
            out_ref[0, h:h + _HCHUNK] = (out_ref[0, h:h + _HCHUNK] * i3
                                         + src3[h:h + _HCHUNK] * p3)


def _body(blocks_ref, x_ref, wmat_ref, s_ref, out_ref, pw_ref, l_ref):
    k = pl.program_id(1)
    s_sum = s_ref[0, 0]

    @pl.when(k == 0)
    def _init():
        _apply(x_ref.at[0], wmat_ref, s_sum, out_ref, pw_ref.at[0], l_ref,
               "init")

    @pl.when((k > 0) & (k < _NSTEP - 1))
    def _mid():
        _apply(blocks_ref.at[0, 0], wmat_ref, s_sum, out_ref, pw_ref.at[0],
               l_ref, "mid")
        _apply(blocks_ref.at[1, 0], wmat_ref, s_sum, out_ref, pw_ref.at[1],
               l_ref, "mid")

    @pl.when(k == _NSTEP - 1)
    def _last():
        _apply(blocks_ref.at[0, 0], wmat_ref, s_sum, out_ref, pw_ref.at[0],
               l_ref, "mid")
        _apply(blocks_ref.at[1, 0], wmat_ref, s_sum, out_ref, pw_ref.at[1],
               l_ref, "last")


def kernel(blocks, x, w, gn_weight, gn_bias):
    del gn_bias
    weff = (w * gn_weight).astype(jnp.float32)
    bt = jnp.transpose(blocks, (0, 1, 3, 4, 2))
    xt = jnp.transpose(x, (0, 2, 3, 1))
    wmat = jnp.broadcast_to(weff[:, None], (_C, 128)).astype(jnp.bfloat16)
    s_sum = jnp.sum(weff).reshape(1, 1)

    out_t = pl.pallas_call(
        _body,
        grid=(_B, _NSTEP),
        in_specs=[
            pl.BlockSpec((2, 1, _H, _W, _C),
                         lambda b, k: (jnp.maximum(k - 1, 0), b, 0, 0, 0)),
            pl.BlockSpec((1, _H, _W, _C),
                         lambda b, k: (jnp.minimum(b + (k >= 3), _B - 1),
                                       0, 0, 0)),
            pl.BlockSpec((_C, 128), lambda b, k: (0, 0)),
            pl.BlockSpec(memory_space=pltpu.SMEM),
        ],
        out_specs=pl.BlockSpec((1, _H, _W, _C), lambda b, k: (b, 0, 0, 0)),
        out_shape=jax.ShapeDtypeStruct((_B, _H, _W, _C), jnp.float32),
        scratch_shapes=[
            pltpu.VMEM((2, _HW, 128), jnp.float32),
            pltpu.VMEM((_HW, 128), jnp.float32),
        ],
        compiler_params=pltpu.CompilerParams(
            dimension_semantics=("parallel", "arbitrary"),
            vmem_limit_bytes=100 * 1024 * 1024,
        ),
    )(bt, xt, wmat, s_sum)
    return jnp.transpose(out_t, (0, 3, 1, 2))


# speedup vs baseline: 1.1615x; 1.0098x over previous
import jax
import jax.numpy as jnp
from jax import lax
from jax.experimental import pallas as pl
from jax.experimental.pallas import tpu as pltpu

_EPS = 1e-5
_N, _B, _C, _H, _W = 8, 4, 256, 64, 64
_NSTEP = _N // 4 + 1
_HW = _H * _W
_INV_CHW = 1.0 / (_C * _H * _W)
_LOG2E = 1.4426950408889634
_HCHUNK = 8


def _stats_and_pw(ref3, wmat_ref, pw_ref):
    acc_s = jnp.zeros((_HCHUNK * _W, _C), jnp.float32)
    acc_q = jnp.zeros((_HCHUNK * _W, _C), jnp.float32)
    for h in range(0, _H, _HCHUNK):
        vc = ref3[h:h + _HCHUNK].reshape(_HCHUNK * _W, _C)
        acc_s = acc_s + vc
        acc_q = acc_q + vc * vc
        pw_ref[h * _W:(h + _HCHUNK) * _W] = jnp.dot(
            vc.astype(jnp.bfloat16), wmat_ref[...],
            preferred_element_type=jnp.float32)
    mean = jnp.sum(acc_s) * _INV_CHW
    var = jnp.sum(acc_q) * _INV_CHW - mean * mean
    return mean, lax.rsqrt(var + _EPS) * _LOG2E


def _apply(src3, wmat_ref, s_sum, out_ref, pw_ref, l_ref, mode):
    mean, rstd = _stats_and_pw(src3, wmat_ref, pw_ref)
    shift = mean * s_sum
    for h in range(0, _H, _HCHUNK):
        r0, r1 = h * _W, (h + _HCHUNK) * _W
        p = jnp.exp2((pw_ref[r0:r1] - shift) * rstd)
        if mode == "init":
            l_ref[r0:r1] = p
            p3 = pltpu.repeat(p, 2, axis=1).reshape(_HCHUNK, _W, _C)
            out_ref[0, h:h + _HCHUNK] = src3[h:h + _HCHUNK] * p3
        elif mode == "mid":
            l_ref[r0:r1] = l_ref[r0:r1] + p
            p3 = pltpu.repeat(p, 2, axis=1).reshape(_HCHUNK, _W, _C)
            out_ref[0, h:h + _HCHUNK] = (out_ref[0, h:h + _HCHUNK]
                                         + src3[h:h + _HCHUNK] * p3)
        else:
            inv_l = 1.0 / (l_ref[r0:r1] + p)
            p3 = pltpu.repeat(p * inv_l, 2, axis=1).reshape(_HCHUNK, _W, _C)
            i3 = pltpu.repeat(inv_l, 2, axis=1).reshape(_HCHUNK, _W, _C)
            out_ref[0, h:h + _HCHUNK] = (out_ref[0, h:h + _HCHUNK] * i3
                                         + src3[h:h + _HCHUNK] * p3)


def _body(blocks_ref, x_ref, wmat_ref, s_ref, out_ref, pw_ref, l_ref):
    k = pl.program_id(1)
    s_sum = s_ref[0, 0]

    @pl.when(k == 0)
    def _init():
        _apply(x_ref.at[0], wmat_ref, s_sum, out_ref, pw_ref.at[0], l_ref,
               "init")

    @pl.when((k > 0) & (k < _NSTEP - 1))
    def _mid():
        for j in range(4):
            _apply(blocks_ref.at[j, 0], wmat_ref, s_sum, out_ref,
                   pw_ref.at[j % 2], l_ref, "mid")

    @pl.when(k == _NSTEP - 1)
    def _last():
        for j in range(3):
            _apply(blocks_ref.at[j, 0], wmat_ref, s_sum, out_ref,
                   pw_ref.at[j % 2], l_ref, "mid")
        _apply(blocks_ref.at[3, 0], wmat_ref, s_sum, out_ref, pw_ref.at[1],
               l_ref, "last")


def kernel(blocks, x, w, gn_weight, gn_bias):
    del gn_bias
    weff = (w * gn_weight).astype(jnp.float32)
    bt = jnp.transpose(blocks, (0, 1, 3, 4, 2))
    xt = jnp.transpose(x, (0, 2, 3, 1))
    wmat = jnp.broadcast_to(weff[:, None], (_C, 128)).astype(jnp.bfloat16)
    s_sum = jnp.sum(weff).reshape(1, 1)

    out_t = pl.pallas_call(
        _body,
        grid=(_B, _NSTEP),
        in_specs=[
            pl.BlockSpec((4, 1, _H, _W, _C),
                         lambda b, k: (jnp.maximum(k - 1, 0), b, 0, 0, 0)),
            pl.BlockSpec((1, _H, _W, _C),
                         lambda b, k: (jnp.minimum(b + (k >= 2), _B - 1),
                                       0, 0, 0)),
            pl.BlockSpec((_C, 128), lambda b, k: (0, 0)),
            pl.BlockSpec(memory_space=pltpu.SMEM),
        ],
        out_specs=pl.BlockSpec((1, _H, _W, _C), lambda b, k: (b, 0, 0, 0)),
        out_shape=jax.ShapeDtypeStruct((_B, _H, _W, _C), jnp.float32),
        scratch_shapes=[
            pltpu.VMEM((2, _HW, 128), jnp.float32),
            pltpu.VMEM((_HW, 128), jnp.float32),
        ],
        compiler_params=pltpu.CompilerParams(
            dimension_semantics=("parallel", "arbitrary"),
            vmem_limit_bytes=100 * 1024 * 1024,
        ),
    )(bt, xt, wmat, s_sum)
    return jnp.transpose(out_t, (0, 3, 1, 2))
